# Initial kernel scaffold; baseline (speedup 1.0000x reference)
import jax
import jax.numpy as jnp
from jax.experimental import pallas as pl


def kernel(x, edge_index, W1l, b1l, W1r, W2l, b2l, W2r):
    raise NotImplementedError("write your pallas kernel here")


# speedup vs baseline: 9.8432x; 9.8432x over previous
import jax
import jax.numpy as jnp
from jax import lax
from jax.experimental import pallas as pl
from jax.experimental.pallas import tpu as pltpu
from jax.experimental.pallas import tpu_sc as plsc

_N = 10000
_E = 320000
_DIN = 128
_DHID = 64
_DOUT = 4

_NC = 2
_NS = 16
_NW = _NC * _NS
_EPW = _E // _NW
_C = 80
_NCHUNK = _EPW // _C
_NP = 10240
_RPT = _NP // _NS
_CW = 16
_F2 = 16


def _edge_pass(F, with_count):
  mesh = plsc.VectorSubcoreMesh(core_axis_name="c", subcore_axis_name="s")
  if with_count:
    out_type = (jax.ShapeDtypeStruct((_NC, _NP, F), jnp.float32),
                jax.ShapeDtypeStruct((_NC, _NP, _CW), jnp.float32))
  else:
    out_type = jax.ShapeDtypeStruct((_NC, _NP, F), jnp.float32)
  scratch = [
      pltpu.VMEM((_NCHUNK, _C), jnp.int32),
      pltpu.VMEM((_NCHUNK, _C), jnp.int32),
      pltpu.VMEM((_C, F), jnp.float32),
      pltpu.VMEM_SHARED((_NP, F), jnp.float32),
      pltpu.SemaphoreType.DMA,
  ]
  if with_count:
    scratch += [
        pltpu.VMEM((_C, _CW), jnp.float32),
        pltpu.VMEM_SHARED((_NP, _CW), jnp.float32),
    ]

  def body(*refs):
    if with_count:
      (y_hbm, src_hbm, dst_hbm, zf_hbm, zc_hbm, ones_hbm,
       out_hbm, cnt_hbm, sidx, didx, msg, acc, sem, onesv, cacc) = refs
    else:
      (y_hbm, src_hbm, dst_hbm, zf_hbm,
       out_hbm, sidx, didx, msg, acc, sem) = refs
    c = lax.axis_index("c")
    s = lax.axis_index("s")
    wid = c * _NS + s
    rows = pl.ds(s * _RPT, _RPT)
    pltpu.sync_copy(zf_hbm.at[rows], acc.at[rows])
    pltpu.sync_copy(src_hbm.at[wid], sidx)
    pltpu.sync_copy(dst_hbm.at[wid], didx)
    if with_count:
      pltpu.sync_copy(zc_hbm.at[rows], cacc.at[rows])
      pltpu.sync_copy(ones_hbm, onesv)
    plsc.subcore_barrier()

    def chunk(j, carry):
      pltpu.async_copy(y_hbm.at[sidx.at[j]], msg, sem).wait()
      pltpu.sync_copy(msg, acc.at[didx.at[j]], add=True)
      if with_count:
        pltpu.sync_copy(onesv, cacc.at[didx.at[j]], add=True)
      return carry

    lax.fori_loop(0, _NCHUNK, chunk, 0)
    plsc.subcore_barrier()
    pltpu.sync_copy(acc.at[rows], out_hbm.at[c].at[rows])
    if with_count:
      pltpu.sync_copy(cacc.at[rows], cnt_hbm.at[c].at[rows])

  return pl.kernel(
      body, mesh=mesh, out_type=out_type, scratch_types=scratch,
      compiler_params=pltpu.CompilerParams(use_tc_tiling_on_sc=False))


_pass1 = _edge_pass(_DHID, with_count=True)
_pass2 = _edge_pass(_F2, with_count=False)


def _mm1_body(x_ref, w_ref, y1_ref, xr_ref):
  o = jnp.dot(x_ref[...], w_ref[...], preferred_element_type=jnp.float32)
  y1_ref[...] = o[:, :_DHID]
  xr_ref[...] = o[:, _DHID:]


_mm1 = pl.pallas_call(
    _mm1_body,
    out_shape=(jax.ShapeDtypeStruct((_N, _DHID), jnp.float32),
               jax.ShapeDtypeStruct((_N, _DHID), jnp.float32)),
)


def _mid_body(s1p_ref, cp_ref, xr_ref, b1_ref, w2_ref, y2_ref, hr2_ref):
  sp = s1p_ref[...]
  s = sp[0, :_N] + sp[1, :_N]
  cps = cp_ref[...]
  cnt = jnp.maximum(cps[0, :_N, :1] + cps[1, :_N, :1], 1.0)
  o = s / cnt + b1_ref[...] + xr_ref[...]
  nrm = jnp.sqrt(jnp.sum(o * o, axis=-1, keepdims=True))
  o = o / jnp.maximum(nrm, 1e-12)
  h = jnp.maximum(o, 0.0)
  p = jnp.dot(h, w2_ref[...], preferred_element_type=jnp.float32)
  y2_ref[...] = p[:, :_F2]
  hr2_ref[...] = p[:, _F2:]


_mid = pl.pallas_call(
    _mid_body,
    out_shape=(jax.ShapeDtypeStruct((_N, _F2), jnp.float32),
               jax.ShapeDtypeStruct((_N, _F2), jnp.float32)),
)


def _fin_body(s2p_ref, cp_ref, hr2_ref, b2_ref, o_ref):
  sp = s2p_ref[...]
  s = sp[0, :_N] + sp[1, :_N]
  cps = cp_ref[...]
  cnt = jnp.maximum(cps[0, :_N, :1] + cps[1, :_N, :1], 1.0)
  o = s / cnt + b2_ref[...] + hr2_ref[...]
  nrm = jnp.sqrt(jnp.sum(o * o, axis=-1, keepdims=True))
  o = o / jnp.maximum(nrm, 1e-12)
  o_ref[...] = o[:, :_DOUT]


_fin = pl.pallas_call(
    _fin_body,
    out_shape=jax.ShapeDtypeStruct((_N, _DOUT), jnp.float32),
)


def kernel(x, edge_index, W1l, b1l, W1r, W2l, b2l, W2r):
  src3 = edge_index[0].reshape(_NW, _NCHUNK, _C)
  dst3 = edge_index[1].reshape(_NW, _NCHUNK, _C)
  w1 = jnp.concatenate([W1l.T, W1r.T], axis=1)
  w2l = jnp.zeros((_DHID, _F2), jnp.float32).at[:, :_DOUT].set(W2l.T)
  w2r = jnp.zeros((_DHID, _F2), jnp.float32).at[:, :_DOUT].set(W2r.T)
  w2 = jnp.concatenate([w2l, w2r], axis=1)
  zf1 = jnp.zeros((_NP, _DHID), jnp.float32)
  zc = jnp.zeros((_NP, _CW), jnp.float32)
  zf2 = jnp.zeros((_NP, _F2), jnp.float32)
  onesb = jnp.ones((_C, _CW), jnp.float32)
  b1 = b1l.reshape(1, _DHID)
  b2 = jnp.zeros((1, _F2), jnp.float32).at[0, :_DOUT].set(b2l)

  y1, xr1 = _mm1(x, w1)
  s1p, cp = _pass1(y1, src3, dst3, zf1, zc, onesb)
  y2p, hr2 = _mid(s1p, cp, xr1, b1, w2)
  s2p = _pass2(y2p, src3, dst3, zf2)
  return _fin(s2p, cp, hr2, b2)


# speedup vs baseline: 18.4155x; 1.8709x over previous
import jax
import jax.numpy as jnp
from jax import lax
from jax.experimental import pallas as pl
from jax.experimental.pallas import tpu as pltpu
from jax.experimental.pallas import tpu_sc as plsc

_N = 10000
_E = 320000
_DIN = 128
_DHID = 64
_DOUT = 4

_NC = 2
_NS = 16
_NW = _NC * _NS
_EPW = _E // _NW
_C = 80
_NCHUNK = _EPW // _C
_B = 5
_NP = 10240
_RPT = _NP // _NS
_CW = 16
_F2 = 16


def _edge_pass(F, with_count):
  mesh = plsc.VectorSubcoreMesh(core_axis_name="c", subcore_axis_name="s")
  if with_count:
    out_type = (jax.ShapeDtypeStruct((_NC, _NP, F), jnp.float32),
                jax.ShapeDtypeStruct((_NC, _NP, _CW), jnp.float32))
  else:
    out_type = jax.ShapeDtypeStruct((_NC, _NP, F), jnp.float32)
  scratch = [
      pltpu.VMEM((_NCHUNK, _C), jnp.int32),
      pltpu.VMEM((_NCHUNK, _C), jnp.int32),
      pltpu.VMEM((_B, _C, F), jnp.float32),
      pltpu.VMEM_SHARED((_NP, F), jnp.float32),
  ] + [pltpu.SemaphoreType.DMA] * _B
  if with_count:
    scratch += [
        pltpu.VMEM((_C, _CW), jnp.float32),
        pltpu.VMEM_SHARED((_NP, _CW), jnp.float32),
    ]

  def body(*refs):
    if with_count:
      (y_hbm, src_hbm, dst_hbm, zf_hbm, zc_hbm, ones_hbm, out_hbm, cnt_hbm,
       sidx, didx, msg, acc, *rest) = refs
      sems = rest[:_B]
      onesv, cacc = rest[_B], rest[_B + 1]
    else:
      (y_hbm, src_hbm, dst_hbm, zf_hbm, out_hbm,
       sidx, didx, msg, acc, *sems) = refs
    c = lax.axis_index("c")
    s = lax.axis_index("s")
    wid = c * _NS + s
    rows = pl.ds(s * _RPT, _RPT)
    pltpu.sync_copy(zf_hbm.at[rows], acc.at[rows])
    pltpu.sync_copy(src_hbm.at[wid], sidx)
    pltpu.sync_copy(dst_hbm.at[wid], didx)
    if with_count:
      pltpu.sync_copy(zc_hbm.at[rows], cacc.at[rows])
      pltpu.sync_copy(ones_hbm, onesv)
    plsc.subcore_barrier()

    for b in range(_B):
      pltpu.async_copy(y_hbm.at[sidx.at[b]], msg.at[b], sems[b])

    def group(g, carry):
      for b in range(_B):
        j = g * _B + b
        pltpu.make_async_copy(y_hbm.at[sidx.at[j]], msg.at[b], sems[b]).wait()
        pltpu.sync_copy(msg.at[b], acc.at[didx.at[j]], add=True)
        if with_count:
          pltpu.sync_copy(onesv, cacc.at[didx.at[j]], add=True)

        @pl.when(j + _B < _NCHUNK)
        def _():
          pltpu.async_copy(y_hbm.at[sidx.at[j + _B]], msg.at[b], sems[b])
      return carry

    lax.fori_loop(0, _NCHUNK // _B, group, 0)
    plsc.subcore_barrier()
    pltpu.sync_copy(acc.at[rows], out_hbm.at[c].at[rows])
    if with_count:
      pltpu.sync_copy(cacc.at[rows], cnt_hbm.at[c].at[rows])

  return pl.kernel(
      body, mesh=mesh, out_type=out_type, scratch_types=scratch,
      compiler_params=pltpu.CompilerParams(use_tc_tiling_on_sc=False))


_pass1 = _edge_pass(_DHID, with_count=True)
_pass2 = _edge_pass(_F2, with_count=False)


def _mm1_body(x_ref, w_ref, y1_ref, xr_ref):
  o = jnp.dot(x_ref[...], w_ref[...], preferred_element_type=jnp.float32)
  y1_ref[...] = o[:, :_DHID]
  xr_ref[...] = o[:, _DHID:]


_mm1 = pl.pallas_call(
    _mm1_body,
    out_shape=(jax.ShapeDtypeStruct((_N, _DHID), jnp.float32),
               jax.ShapeDtypeStruct((_N, _DHID), jnp.float32)),
)


def _mid_body(s1p_ref, cp_ref, xr_ref, b1_ref, w2_ref, y2_ref, hr2_ref):
  sp = s1p_ref[...]
  s = sp[0, :_N] + sp[1, :_N]
  cps = cp_ref[...]
  cnt = jnp.maximum(cps[0, :_N, :1] + cps[1, :_N, :1], 1.0)
  o = s / cnt + b1_ref[...] + xr_ref[...]
  nrm = jnp.sqrt(jnp.sum(o * o, axis=-1, keepdims=True))
  o = o / jnp.maximum(nrm, 1e-12)
  h = jnp.maximum(o, 0.0)
  p = jnp.dot(h, w2_ref[...], preferred_element_type=jnp.float32)
  y2_ref[...] = p[:, :_F2]
  hr2_ref[...] = p[:, _F2:]


_mid = pl.pallas_call(
    _mid_body,
    out_shape=(jax.ShapeDtypeStruct((_N, _F2), jnp.float32),
               jax.ShapeDtypeStruct((_N, _F2), jnp.float32)),
)


def _fin_body(s2p_ref, cp_ref, hr2_ref, b2_ref, o_ref):
  sp = s2p_ref[...]
  s = sp[0, :_N] + sp[1, :_N]
  cps = cp_ref[...]
  cnt = jnp.maximum(cps[0, :_N, :1] + cps[1, :_N, :1], 1.0)
  o = s / cnt + b2_ref[...] + hr2_ref[...]
  nrm = jnp.sqrt(jnp.sum(o * o, axis=-1, keepdims=True))
  o = o / jnp.maximum(nrm, 1e-12)
  o_ref[...] = o[:, :_DOUT]


_fin = pl.pallas_call(
    _fin_body,
    out_shape=jax.ShapeDtypeStruct((_N, _DOUT), jnp.float32),
)


def kernel(x, edge_index, W1l, b1l, W1r, W2l, b2l, W2r):
  src3 = edge_index[0].reshape(_NW, _NCHUNK, _C)
  dst3 = edge_index[1].reshape(_NW, _NCHUNK, _C)
  w1 = jnp.concatenate([W1l.T, W1r.T], axis=1)
  w2l = jnp.zeros((_DHID, _F2), jnp.float32).at[:, :_DOUT].set(W2l.T)
  w2r = jnp.zeros((_DHID, _F2), jnp.float32).at[:, :_DOUT].set(W2r.T)
  w2 = jnp.concatenate([w2l, w2r], axis=1)
  zf1 = jnp.zeros((_NP, _DHID), jnp.float32)
  zc = jnp.zeros((_NP, _CW), jnp.float32)
  zf2 = jnp.zeros((_NP, _F2), jnp.float32)
  onesb = jnp.ones((_C, _CW), jnp.float32)
  b1 = b1l.reshape(1, _DHID)
  b2 = jnp.zeros((1, _F2), jnp.float32).at[0, :_DOUT].set(b2l)

  y1, xr1 = _mm1(x, w1)
  s1p, cp = _pass1(y1, src3, dst3, zf1, zc, onesb)
  y2p, hr2 = _mid(s1p, cp, xr1, b1, w2)
  s2p = _pass2(y2p, src3, dst3, zf2)
  return _fin(s2p, cp, hr2, b2)


# speedup vs baseline: 18.8433x; 1.0232x over previous
import jax
import jax.numpy as jnp
from jax import lax
from jax.experimental import pallas as pl
from jax.experimental.pallas import tpu as pltpu
from jax.experimental.pallas import tpu_sc as plsc

_N = 10000
_E = 320000
_DIN = 128
_DHID = 64
_DOUT = 4

_NC = 2
_NS = 16
_NW = _NC * _NS
_EPW = _E // _NW
_C = 80
_NCHUNK = _EPW // _C
_B = 5
_NP = 10240
_RPT = _NP // _NS
_CW = 8
_F2 = 16


def _edge_pass(F, with_count):
  mesh = plsc.VectorSubcoreMesh(core_axis_name="c", subcore_axis_name="s")
  if with_count:
    out_type = (jax.ShapeDtypeStruct((_NC, _NP, F), jnp.float32),
                jax.ShapeDtypeStruct((_NC, _NP, _CW), jnp.float32))
  else:
    out_type = jax.ShapeDtypeStruct((_NC, _NP, F), jnp.float32)
  scratch = [
      pltpu.VMEM((_NCHUNK, _C), jnp.int32),
      pltpu.VMEM((_NCHUNK, _C), jnp.int32),
      pltpu.VMEM((_B, _C, F), jnp.float32),
      pltpu.VMEM_SHARED((_NP, F), jnp.float32),
  ] + [pltpu.SemaphoreType.DMA] * (2 * _B)
  if with_count:
    scratch += [
        pltpu.VMEM((_C, _CW), jnp.float32),
        pltpu.VMEM_SHARED((_NP, _CW), jnp.float32),
    ]

  def body(*refs):
    if with_count:
      (y_hbm, src_hbm, dst_hbm, zf_hbm, zc_hbm, ones_hbm, out_hbm, cnt_hbm,
       sidx, didx, msg, acc, *rest) = refs
      gsem, ssem = rest[:_B], rest[_B:2 * _B]
      onesv, cacc = rest[2 * _B], rest[2 * _B + 1]
    else:
      (y_hbm, src_hbm, dst_hbm, zf_hbm, out_hbm,
       sidx, didx, msg, acc, *rest) = refs
      gsem, ssem = rest[:_B], rest[_B:2 * _B]
    c = lax.axis_index("c")
    s = lax.axis_index("s")
    wid = c * _NS + s
    rows = pl.ds(s * _RPT, _RPT)
    pltpu.sync_copy(zf_hbm.at[rows], acc.at[rows])
    pltpu.sync_copy(src_hbm.at[wid], sidx)
    pltpu.sync_copy(dst_hbm.at[wid], didx)
    if with_count:
      pltpu.sync_copy(zc_hbm.at[rows], cacc.at[rows])
      pltpu.sync_copy(ones_hbm, onesv)
    plsc.subcore_barrier()

    for b in range(_B):
      pltpu.async_copy(y_hbm.at[sidx.at[b]], msg.at[b], gsem[b])

    def drain_scatter(pb, j):
      pltpu.make_async_copy(msg.at[pb], acc.at[didx.at[j]], ssem[pb]).wait()
      if with_count:
        pltpu.make_async_copy(onesv, cacc.at[didx.at[j]], ssem[pb]).wait()

    def group(g, carry):
      for b in range(_B):
        j = g * _B + b
        pltpu.make_async_copy(y_hbm.at[sidx.at[j]], msg.at[b], gsem[b]).wait()
        pltpu.async_copy(msg.at[b], acc.at[didx.at[j]], ssem[b], add=True)
        if with_count:
          pltpu.async_copy(onesv, cacc.at[didx.at[j]], ssem[b], add=True)
        pb = (b - 1) % _B

        @pl.when(j >= 1)
        def _():
          drain_scatter(pb, j)

          @pl.when(j - 1 + _B < _NCHUNK)
          def _():
            pltpu.async_copy(y_hbm.at[sidx.at[j - 1 + _B]], msg.at[pb],
                             gsem[pb])
      return carry

    lax.fori_loop(0, _NCHUNK // _B, group, 0)
    drain_scatter(_B - 1, _NCHUNK - 1)
    plsc.subcore_barrier()
    pltpu.sync_copy(acc.at[rows], out_hbm.at[c].at[rows])
    if with_count:
      pltpu.sync_copy(cacc.at[rows], cnt_hbm.at[c].at[rows])

  return pl.kernel(
      body, mesh=mesh, out_type=out_type, scratch_types=scratch,
      compiler_params=pltpu.CompilerParams(use_tc_tiling_on_sc=False))


_pass1 = _edge_pass(_DHID, with_count=True)
_pass2 = _edge_pass(_F2, with_count=False)


def _mm1_body(x_ref, w_ref, y1_ref, xr_ref):
  o = jnp.dot(x_ref[...], w_ref[...], preferred_element_type=jnp.float32)
  y1_ref[...] = o[:, :_DHID]
  xr_ref[...] = o[:, _DHID:]


_mm1 = pl.pallas_call(
    _mm1_body,
    out_shape=(jax.ShapeDtypeStruct((_N, _DHID), jnp.float32),
               jax.ShapeDtypeStruct((_N, _DHID), jnp.float32)),
)


def _mid_body(s1p_ref, cp_ref, xr_ref, b1_ref, w2_ref, y2_ref, hr2_ref):
  sp = s1p_ref[...]
  s = sp[0, :_N] + sp[1, :_N]
  cps = cp_ref[...]
  cnt = jnp.maximum(cps[0, :_N, :1] + cps[1, :_N, :1], 1.0)
  o = s / cnt + b1_ref[...] + xr_ref[...]
  nrm = jnp.sqrt(jnp.sum(o * o, axis=-1, keepdims=True))
  o = o / jnp.maximum(nrm, 1e-12)
  h = jnp.maximum(o, 0.0)
  p = jnp.dot(h, w2_ref[...], preferred_element_type=jnp.float32)
  y2_ref[...] = p[:, :_F2]
  hr2_ref[...] = p[:, _F2:]


_mid = pl.pallas_call(
    _mid_body,
    out_shape=(jax.ShapeDtypeStruct((_N, _F2), jnp.float32),
               jax.ShapeDtypeStruct((_N, _F2), jnp.float32)),
)


def _fin_body(s2p_ref, cp_ref, hr2_ref, b2_ref, o_ref):
  sp = s2p_ref[...]
  s = sp[0, :_N] + sp[1, :_N]
  cps = cp_ref[...]
  cnt = jnp.maximum(cps[0, :_N, :1] + cps[1, :_N, :1], 1.0)
  o = s / cnt + b2_ref[...] + hr2_ref[...]
  nrm = jnp.sqrt(jnp.sum(o * o, axis=-1, keepdims=True))
  o = o / jnp.maximum(nrm, 1e-12)
  o_ref[...] = o[:, :_DOUT]


_fin = pl.pallas_call(
    _fin_body,
    out_shape=jax.ShapeDtypeStruct((_N, _DOUT), jnp.float32),
)


def kernel(x, edge_index, W1l, b1l, W1r, W2l, b2l, W2r):
  src3 = edge_index[0].reshape(_NW, _NCHUNK, _C)
  dst3 = edge_index[1].reshape(_NW, _NCHUNK, _C)
  w1 = jnp.concatenate([W1l.T, W1r.T], axis=1)
  w2l = jnp.zeros((_DHID, _F2), jnp.float32).at[:, :_DOUT].set(W2l.T)
  w2r = jnp.zeros((_DHID, _F2), jnp.float32).at[:, :_DOUT].set(W2r.T)
  w2 = jnp.concatenate([w2l, w2r], axis=1)
  zf1 = jnp.zeros((_NP, _DHID), jnp.float32)
  zc = jnp.zeros((_NP, _CW), jnp.float32)
  zf2 = jnp.zeros((_NP, _F2), jnp.float32)
  onesb = jnp.ones((_C, _CW), jnp.float32)
  b1 = b1l.reshape(1, _DHID)
  b2 = jnp.zeros((1, _F2), jnp.float32).at[0, :_DOUT].set(b2l)

  y1, xr1 = _mm1(x, w1)
  s1p, cp = _pass1(y1, src3, dst3, zf1, zc, onesb)
  y2p, hr2 = _mid(s1p, cp, xr1, b1, w2)
  s2p = _pass2(y2p, src3, dst3, zf2)
  return _fin(s2p, cp, hr2, b2)


# speedup vs baseline: 20.6951x; 1.0983x over previous
import jax
import jax.numpy as jnp
from jax import lax
from jax.experimental import pallas as pl
from jax.experimental.pallas import tpu as pltpu
from jax.experimental.pallas import tpu_sc as plsc

_N = 10000
_E = 320000
_DIN = 128
_DHID = 64
_DOUT = 4

_NC = 2
_NS = 16
_NW = _NC * _NS
_EPW = _E // _NW
_C = 80
_NCHUNK = _EPW // _C
_B = 5
_NP = 10240
_RPT = _NP // _NS
_EP = 320
_F2 = 16

_mesh = plsc.VectorSubcoreMesh(core_axis_name="c", subcore_axis_name="s")
_sc_params = pltpu.CompilerParams(use_tc_tiling_on_sc=False)


def _stage_and_loop(y_hbm, sidx, didx, msg, acc, gsem, ssem, F,
                    extra_scatter=None):
  for b in range(_B):
    pltpu.async_copy(y_hbm.at[sidx.at[b]], msg.at[b], gsem[b])

  def drain_scatter(pb, j):
    pltpu.make_async_copy(msg.at[pb], acc.at[didx.at[j]], ssem[pb]).wait()
    if extra_scatter is not None:
      extra_scatter.drain(pb, j)

  def group(g, carry):
    for b in range(_B):
      j = g * _B + b
      pltpu.make_async_copy(y_hbm.at[sidx.at[j]], msg.at[b], gsem[b]).wait()
      pltpu.async_copy(msg.at[b], acc.at[didx.at[j]], ssem[b], add=True)
      if extra_scatter is not None:
        extra_scatter.issue(b, j)
      pb = (b - 1) % _B

      @pl.when(j >= 1)
      def _():
        drain_scatter(pb, j)

        @pl.when(j - 1 + _B < _NCHUNK)
        def _():
          pltpu.async_copy(y_hbm.at[sidx.at[j - 1 + _B]], msg.at[pb],
                           gsem[pb])
    return carry

  lax.fori_loop(0, _NCHUNK // _B, group, 0)
  drain_scatter(_B - 1, _NCHUNK - 1)


class _CountScatter:
  def __init__(self, onesv, cacc, didx, didx2, ssem):
    self.onesv, self.cacc, self.didx, self.didx2, self.ssem = (
        onesv, cacc, didx, didx2, ssem)

  def issue(self, b, j):
    pltpu.async_copy(self.onesv, self.cacc.at[self.didx.at[j]],
                     self.ssem[b], add=True)
    pltpu.async_copy(self.onesv, self.cacc.at[self.didx2.at[j]],
                     self.ssem[b], add=True)

  def drain(self, pb, j):
    pltpu.make_async_copy(self.onesv, self.cacc.at[self.didx.at[j]],
                          self.ssem[pb]).wait()
    pltpu.make_async_copy(self.onesv, self.cacc.at[self.didx2.at[j]],
                          self.ssem[pb]).wait()


def _make_pass1():
  out_type = (jax.ShapeDtypeStruct((_NC, _NP, _DHID), jnp.float32),
              jax.ShapeDtypeStruct((_NP,), jnp.float32))
  scratch = [
      pltpu.VMEM((_NCHUNK, _C), jnp.int32),
      pltpu.VMEM((_NCHUNK, _C), jnp.int32),
      pltpu.VMEM((_NCHUNK, _C), jnp.int32),
      pltpu.VMEM((_B, _C, _DHID), jnp.float32),
      pltpu.VMEM((_C,), jnp.float32),
      pltpu.VMEM((_EP, _DHID), jnp.float32),
      pltpu.VMEM((_EP,), jnp.float32),
      pltpu.VMEM((_EP,), jnp.float32),
      pltpu.VMEM_SHARED((_NP, _DHID), jnp.float32),
      pltpu.VMEM_SHARED((_NP,), jnp.float32),
  ] + [pltpu.SemaphoreType.DMA] * (2 * _B)

  def body(y_hbm, ei_hbm, zf_hbm, zc_hbm, ones_hbm, out_hbm, rcnt_hbm,
           sidx, didx, didx2, msg, onesv, eb, cb, rb, acc, cacc, *sems):
    gsem, ssem = sems[:_B], sems[_B:]
    c = lax.axis_index("c")
    s = lax.axis_index("s")
    wid = c * _NS + s
    wid2 = (1 - c) * _NS + s
    rows = pl.ds(s * _RPT, _RPT)
    pltpu.sync_copy(zf_hbm.at[rows], acc.at[rows])
    pltpu.sync_copy(zc_hbm.at[rows], cacc.at[rows])
    pltpu.sync_copy(ei_hbm.at[0].at[wid], sidx)
    pltpu.sync_copy(ei_hbm.at[1].at[wid], didx)
    pltpu.sync_copy(ei_hbm.at[1].at[wid2], didx2)
    pltpu.sync_copy(ones_hbm, onesv)
    plsc.subcore_barrier()

    cs = _CountScatter(onesv, cacc, didx, didx2, ssem)
    _stage_and_loop(y_hbm, sidx, didx, msg, acc, gsem, ssem, _DHID,
                    extra_scatter=cs)
    plsc.subcore_barrier()

    for half in range(2):
      rbase = s * _RPT + half * _EP
      strip = pl.ds(rbase, _EP)
      pltpu.sync_copy(acc.at[strip], eb)
      pltpu.sync_copy(cacc.at[strip], cb)

      for k in range(_EP // 16):
        lanes = pl.ds(16 * k, 16)
        rb[lanes] = 1.0 / jnp.maximum(cb[lanes], 1.0)

      def egroup(g, carry):
        cvec = rb[pl.ds(g * 16, 16)]
        for l in range(16):
          rr = g * 16 + l
          rv = cvec[l]
          for k in range(_DHID // 16):
            col = pl.ds(16 * k, 16)
            eb[rr, col] = eb[rr, col] * rv
        return carry

      lax.fori_loop(0, _EP // 16, egroup, 0)
      pltpu.sync_copy(eb, out_hbm.at[c].at[strip])

      @pl.when(c == 0)
      def _():
        pltpu.sync_copy(rb, rcnt_hbm.at[strip])

  return pl.kernel(body, mesh=_mesh, out_type=out_type,
                   scratch_types=scratch, compiler_params=_sc_params)


def _make_pass2():
  out_type = jax.ShapeDtypeStruct((_NC, _NP, _F2), jnp.float32)
  scratch = [
      pltpu.VMEM((_NCHUNK, _C), jnp.int32),
      pltpu.VMEM((_NCHUNK, _C), jnp.int32),
      pltpu.VMEM((_B, _C, _F2), jnp.float32),
      pltpu.VMEM((_RPT,), jnp.float32),
      pltpu.VMEM((_EP, _F2), jnp.float32),
      pltpu.VMEM_SHARED((_NP, _F2), jnp.float32),
  ] + [pltpu.SemaphoreType.DMA] * (2 * _B)

  def body(y_hbm, ei_hbm, zf_hbm, rcnt_hbm, out_hbm,
           sidx, didx, msg, rstage, eb, acc, *sems):
    gsem, ssem = sems[:_B], sems[_B:]
    c = lax.axis_index("c")
    s = lax.axis_index("s")
    wid = c * _NS + s
    rows = pl.ds(s * _RPT, _RPT)
    pltpu.sync_copy(zf_hbm.at[rows], acc.at[rows])
    pltpu.sync_copy(ei_hbm.at[0].at[wid], sidx)
    pltpu.sync_copy(ei_hbm.at[1].at[wid], didx)
    pltpu.sync_copy(rcnt_hbm.at[rows], rstage)
    plsc.subcore_barrier()

    _stage_and_loop(y_hbm, sidx, didx, msg, acc, gsem, ssem, _F2)
    plsc.subcore_barrier()

    for half in range(2):
      strip = pl.ds(s * _RPT + half * _EP, _EP)
      pltpu.sync_copy(acc.at[strip], eb)

      def egroup(g, carry):
        cvec = rstage[pl.ds(half * _EP + g * 16, 16)]
        for l in range(16):
          rr = g * 16 + l
          eb[rr] = eb[rr] * cvec[l]
        return carry

      lax.fori_loop(0, _EP // 16, egroup, 0)
      pltpu.sync_copy(eb, out_hbm.at[c].at[strip])

  return pl.kernel(body, mesh=_mesh, out_type=out_type,
                   scratch_types=scratch, compiler_params=_sc_params)


_pass1 = _make_pass1()
_pass2 = _make_pass2()


def _mm1_body(x_ref, w_ref, y1_ref, xr_ref):
  o = jnp.dot(x_ref[...], w_ref[...], preferred_element_type=jnp.float32)
  y1_ref[...] = o[:, :_DHID]
  xr_ref[...] = o[:, _DHID:]


_mm1 = pl.pallas_call(
    _mm1_body,
    out_shape=(jax.ShapeDtypeStruct((_N, _DHID), jnp.float32),
               jax.ShapeDtypeStruct((_N, _DHID), jnp.float32)),
)


def _mid_body(s1p_ref, xr_ref, b1_ref, w2_ref, y2_ref, hr2_ref):
  sp = s1p_ref[...]
  o = sp[0, :_N] + sp[1, :_N] + b1_ref[...] + xr_ref[...]
  nrm = jnp.sqrt(jnp.sum(o * o, axis=-1, keepdims=True))
  o = o / jnp.maximum(nrm, 1e-12)
  h = jnp.maximum(o, 0.0)
  p = jnp.dot(h, w2_ref[...], preferred_element_type=jnp.float32)
  y2_ref[...] = p[:, :_F2]
  hr2_ref[...] = p[:, _F2:]


_mid = pl.pallas_call(
    _mid_body,
    out_shape=(jax.ShapeDtypeStruct((_N, _F2), jnp.float32),
               jax.ShapeDtypeStruct((_N, _F2), jnp.float32)),
)


def _fin_body(s2p_ref, hr2_ref, b2_ref, o_ref):
  sp = s2p_ref[...]
  o = sp[0, :_N] + sp[1, :_N] + b2_ref[...] + hr2_ref[...]
  nrm = jnp.sqrt(jnp.sum(o * o, axis=-1, keepdims=True))
  o = o / jnp.maximum(nrm, 1e-12)
  o_ref[...] = o[:, :_DOUT]


_fin = pl.pallas_call(
    _fin_body,
    out_shape=jax.ShapeDtypeStruct((_N, _DOUT), jnp.float32),
)


def kernel(x, edge_index, W1l, b1l, W1r, W2l, b2l, W2r):
  ei = edge_index.reshape(2, _NW, _NCHUNK, _C)
  w1 = jnp.concatenate([W1l.T, W1r.T], axis=1)
  w2l = jnp.zeros((_DHID, _F2), jnp.float32).at[:, :_DOUT].set(W2l.T)
  w2r = jnp.zeros((_DHID, _F2), jnp.float32).at[:, :_DOUT].set(W2r.T)
  w2 = jnp.concatenate([w2l, w2r], axis=1)
  zf1 = jnp.zeros((_NP, _DHID), jnp.float32)
  zc = jnp.zeros((_NP,), jnp.float32)
  zf2 = jnp.zeros((_NP, _F2), jnp.float32)
  onesb = jnp.ones((_C,), jnp.float32)
  b1 = b1l.reshape(1, _DHID)
  b2 = jnp.zeros((1, _F2), jnp.float32).at[0, :_DOUT].set(b2l)

  y1, xr1 = _mm1(x, w1)
  s1p, rcnt = _pass1(y1, ei, zf1, zc, onesb)
  y2p, hr2 = _mid(s1p, xr1, b1, w2)
  s2p = _pass2(y2p, ei, zf2, rcnt)
  return _fin(s2p, hr2, b2)
